# baseline (device time: 28556 ns/iter reference)
import jax
import jax.numpy as jnp
from jax import lax
from jax.experimental import pallas as pl
from jax.experimental.pallas import tpu as pltpu

K = 16
NCHUNK = 2


def _topk_desc(vals, k):
    r, c = vals.shape
    iota = lax.broadcasted_iota(jnp.int32, (r, c), 1)
    cols = []
    work = vals
    for _ in range(k):
        m = jnp.max(work, axis=1, keepdims=True)
        eq = work == m
        first = jnp.min(jnp.where(eq, iota, c), axis=1, keepdims=True)
        work = jnp.where(iota == first, -jnp.inf, work)
        cols.append(m)
    return jnp.concatenate(cols, axis=1)


def _topk_packed(xb, k):
    r, n = xb.shape
    b = pltpu.bitcast(xb, jnp.int32)
    t20 = (b >> 12) & 0xFFFFF
    v = jnp.where(b >= 0, t20, 0x80000 - t20)
    iota = lax.broadcasted_iota(jnp.int32, (r, n), 1)
    keys = v * 4096 + iota
    neg = jnp.iinfo(jnp.int32).min
    cols = []
    for _ in range(k):
        mk = jnp.max(keys, axis=1, keepdims=True)
        keys = jnp.where(keys == mk, neg, keys)
        cols.append(mk)
    mks = jnp.concatenate(cols, axis=1)
    vv = mks >> 12
    t20r = jnp.where(vv >= 0, vv, 0x80000 - vv)
    return pltpu.bitcast(t20r << 12, jnp.float32)


def kernel(x):
    m, n = x.shape
    rh = m // 2
    hc = rh // NCHUNK

    def body(x_hbm, out_ref, xv_ref, loc_ref, xbuf_ref, mrg_ref,
             copy_sems, x_send, x_recv, y_send, y_recv):
        my_x = lax.axis_index("x")
        my_y = lax.axis_index("y")

        barrier_sem = pltpu.get_barrier_semaphore()
        pl.semaphore_signal(barrier_sem, inc=1, device_id=(1 - my_x, my_y),
                            device_id_type=pl.DeviceIdType.MESH)
        pl.semaphore_signal(barrier_sem, inc=1, device_id=(my_x, 1 - my_y),
                            device_id_type=pl.DeviceIdType.MESH)

        copies = []
        for c in range(NCHUNK):
            cp = pltpu.make_async_copy(
                x_hbm.at[pl.ds(my_y * rh + c * hc, hc), :],
                xv_ref.at[c],
                copy_sems.at[c],
            )
            cp.start()
            copies.append(cp)

        def make_x_rdma(c):
            return pltpu.make_async_remote_copy(
                src_ref=loc_ref.at[c], dst_ref=xbuf_ref.at[c],
                send_sem=x_send.at[c], recv_sem=x_recv.at[c],
                device_id=(1 - my_x, my_y),
                device_id_type=pl.DeviceIdType.MESH,
            )

        def make_y_rdma(c):
            return pltpu.make_async_remote_copy(
                src_ref=mrg_ref.at[c],
                dst_ref=out_ref.at[pl.ds(my_y * rh + c * hc, hc), :],
                send_sem=y_send.at[c], recv_sem=y_recv.at[c],
                device_id=(my_x, 1 - my_y),
                device_id_type=pl.DeviceIdType.MESH,
            )

        copies[0].wait()
        loc_ref[0] = _topk_packed(xv_ref[0], K)
        pl.semaphore_wait(barrier_sem, 2)
        rx0 = make_x_rdma(0)
        rx0.start()

        copies[1].wait()
        loc_ref[1] = _topk_packed(xv_ref[1], K)
        rx1 = make_x_rdma(1)
        rx1.start()

        rx0.wait()
        mrg_ref[0] = _topk_desc(
            jnp.concatenate([loc_ref[0], xbuf_ref[0]], axis=1), K)
        ry0 = make_y_rdma(0)
        ry0.start()

        rx1.wait()
        mrg_ref[1] = _topk_desc(
            jnp.concatenate([loc_ref[1], xbuf_ref[1]], axis=1), K)
        ry1 = make_y_rdma(1)
        ry1.start()

        out_ref[pl.ds(my_y * rh, hc), :] = mrg_ref[0]
        out_ref[pl.ds(my_y * rh + hc, hc), :] = mrg_ref[1]

        ry0.wait()
        ry1.wait()

    return pl.pallas_call(
        body,
        out_shape=jax.ShapeDtypeStruct((m, K), jnp.float32),
        in_specs=[pl.BlockSpec(memory_space=pl.ANY)],
        out_specs=pl.BlockSpec(memory_space=pltpu.VMEM),
        scratch_shapes=[
            pltpu.VMEM((NCHUNK, hc, n), jnp.float32),
            pltpu.VMEM((NCHUNK, hc, K), jnp.float32),
            pltpu.VMEM((NCHUNK, hc, K), jnp.float32),
            pltpu.VMEM((NCHUNK, hc, K), jnp.float32),
            pltpu.SemaphoreType.DMA((NCHUNK,)),
            pltpu.SemaphoreType.DMA((NCHUNK,)),
            pltpu.SemaphoreType.DMA((NCHUNK,)),
            pltpu.SemaphoreType.DMA((NCHUNK,)),
            pltpu.SemaphoreType.DMA((NCHUNK,)),
        ],
        compiler_params=pltpu.CompilerParams(collective_id=0),
    )(x)
